# Initial kernel scaffold; baseline (speedup 1.0000x reference)
import jax
import jax.numpy as jnp
from jax.experimental import pallas as pl


def kernel(input_ids, cls_token, W_attn, b_attn, norm_w, in_proj_W, conv_w, conv_b, x_proj_W, dt_w, dt_b, A_log, D_ssm, out_proj_W, head_w, head_b, n_tweets, n_words):
    raise NotImplementedError("write your pallas kernel here")


# speedup vs baseline: 16.5102x; 16.5102x over previous
import jax
import jax.numpy as jnp
from jax.experimental import pallas as pl
from jax.experimental.pallas import tpu as pltpu

B, T, WMAX, DM = 4, 512, 50, 200
DI, DS, DTR, K = 400, 16, 13, 4
POS = T // 2
L = T + 1

TC_AGG = 128

_INTERP = False


def _sigmoid(x):
    return 1.0 / (1.0 + jnp.exp(-x))


def _silu(x):
    return x * _sigmoid(x)


def _softplus(x):
    return jnp.maximum(x, 0.0) + jnp.log(1.0 + jnp.exp(-jnp.abs(x)))


def _agg_body(nw_ref, ids_ref, wa_ref, ba_ref, out_ref):
    ids = ids_ref[0]
    wa = wa_ref[:, 0]
    scores = jnp.sum(ids * wa[None, None, :], axis=-1) + ba_ref[0, 0]
    nw = nw_ref[0, 0]
    wm = jax.lax.broadcasted_iota(jnp.int32, (TC_AGG, WMAX), 1) < nw[:, None]
    scores = jnp.where(wm, scores, -1e30)
    m = jnp.max(scores, axis=1, keepdims=True)
    e = jnp.exp(scores - m)
    attn = e / jnp.sum(e, axis=1, keepdims=True)
    attn = jnp.where(wm, attn, 0.0)
    out_ref[0] = jnp.sum(attn[:, :, None] * ids, axis=1)


def _mamba_body(emb_ref, nt_ref, cls_ref, normw_ref, inW_ref, convw_ref,
                convb_ref, xW_ref, dtw_ref, dtb_ref, Alog_ref, Dssm_ref,
                outW_ref, headw_ref, headb_ref, out_ref,
                dtF_s, duF_s, bcF_s, dtB_s, duB_s, bcB_s):
    convw = convw_ref[...]
    convb = convb_ref[0][None, :]
    dtb = dtb_ref[0][None, :]
    normw = normw_ref[0][None, :]
    cls = cls_ref[...]
    AT = -jnp.exp(jnp.transpose(Alog_ref[...]))

    xsF_pos = []
    xsB_pos = []
    z_pos = []
    for b in range(B):
        nt_b = jnp.broadcast_to(nt_ref[0:1, b:b + 1], (T, DM))
        emb_b = emb_ref[b]
        tmask = jax.lax.broadcasted_iota(jnp.int32, (T, DM), 0) < nt_b
        x0 = jnp.where(tmask, emb_b, 0.0)
        x_b = jnp.concatenate([x0[:POS], cls, x0[POS:]], axis=0)
        h_b = x_b * jax.lax.rsqrt(
            jnp.mean(x_b * x_b, axis=-1, keepdims=True) + 1e-5) * normw
        xz_b = h_b @ inW_ref[...]
        xs0 = xz_b[:, :DI]
        z_pos.append(xz_b[POS:POS + 1, DI:])

        zpad = jnp.zeros((K - 1, DI), jnp.float32)
        xpF = jnp.concatenate([zpad, xs0], axis=0)
        xpB = jnp.concatenate([xs0, zpad], axis=0)
        accF = convb
        accB = convb
        for k in range(K):
            wk = convw[:, k][None, :]
            accF = accF + xpF[k:k + L] * wk
            accB = accB + xpB[K - 1 - k:K - 1 - k + L] * wk
        xsF_b = _silu(accF)
        xsB_b = _silu(accB)
        xsF_pos.append(xsF_b[POS:POS + 1])
        xsB_pos.append(xsB_b[POS:POS + 1])

        for xs_b, dt_s, du_s, bc_s in (
                (xsF_b, dtF_s, duF_s, bcF_s),
                (xsB_b, dtB_s, duB_s, bcB_s)):
            dbc_b = xs_b @ xW_ref[...]
            dt_b = _softplus(dbc_b[:, :DTR] @ dtw_ref[...] + dtb)
            dt_s[b] = dt_b
            du_s[b] = dt_b * xs_b
            bc_s[b] = dbc_b[:, DTR:]

    def dir_step(b, i, hcarry, dt_s, du_s, bc_s):
        dt_t = dt_s[b, pl.ds(i, 1), :]
        du_t = du_s[b, pl.ds(i, 1), :]
        bc_t = bc_s[b, pl.ds(i, 1), :]
        BmT = jnp.transpose(bc_t[:, :DS])
        dA_t = jnp.exp(dt_t * AT)
        return dA_t * hcarry + du_t * BmT

    def step(i, carry):
        new = []
        for b in range(B):
            hf, hb = carry[b]
            hf = dir_step(b, i, hf, dtF_s, duF_s, bcF_s)
            hb = dir_step(b, L - 1 - i, hb, dtB_s, duB_s, bcB_s)
            new.append((hf, hb))
        return tuple(new)

    h0 = jnp.zeros((DS, DI), jnp.float32)
    hfin = jax.lax.fori_loop(0, POS + 1, step, tuple((h0, h0) for _ in range(B)))

    dssm = Dssm_ref[0][None, :]
    grows = []
    for b in range(B):
        hf, hb = hfin[b]
        CmTF = jnp.transpose(bcF_s[b, pl.ds(POS, 1), DS:])
        CmTB = jnp.transpose(bcB_s[b, pl.ds(POS, 1), DS:])
        yF = jnp.sum(hf * CmTF, axis=0, keepdims=True)
        yB = jnp.sum(hb * CmTB, axis=0, keepdims=True)
        g = ((yF + xsF_pos[b] * dssm) + (yB + xsB_pos[b] * dssm)) * _silu(z_pos[b])
        grows.append(g)
    G = jnp.concatenate(grows, axis=0)
    outp = G @ outW_ref[...]
    xfin = jnp.broadcast_to(cls, (B, DM)) + outp
    logits = xfin @ headw_ref[...] + jnp.broadcast_to(headb_ref[0:1, 0:1], (B, 1))
    out_ref[...] = _sigmoid(logits)


def kernel(input_ids, cls_token, W_attn, b_attn, norm_w, in_proj_W, conv_w,
           conv_b, x_proj_W, dt_w, dt_b, A_log, D_ssm, out_proj_W, head_w,
           head_b, n_tweets, n_words):
    nw3 = jnp.reshape(n_words, (B, 1, T)).astype(jnp.int32)
    ba2 = jnp.reshape(b_attn, (1, 1))
    nt2 = jnp.reshape(n_tweets, (1, B)).astype(jnp.int32)
    cb2 = jnp.reshape(conv_b, (1, DI))
    dtb2 = jnp.reshape(dt_b, (1, DI))
    dssm2 = jnp.reshape(D_ssm, (1, DI))
    normw2 = jnp.reshape(norm_w, (1, DM))
    hb2 = jnp.reshape(head_b, (1, 1))

    emb = pl.pallas_call(
        _agg_body,
        grid=(B, T // TC_AGG),
        in_specs=[
            pl.BlockSpec((1, 1, TC_AGG), lambda b, t: (b, 0, t)),
            pl.BlockSpec((1, TC_AGG, WMAX, DM), lambda b, t: (b, t, 0, 0)),
            pl.BlockSpec((DM, 1), lambda b, t: (0, 0)),
            pl.BlockSpec((1, 1), lambda b, t: (0, 0)),
        ],
        out_specs=pl.BlockSpec((1, TC_AGG, DM), lambda b, t: (b, t, 0)),
        out_shape=jax.ShapeDtypeStruct((B, T, DM), jnp.float32),
        interpret=_INTERP,
    )(nw3, input_ids, W_attn, ba2)

    out = pl.pallas_call(
        _mamba_body,
        out_shape=jax.ShapeDtypeStruct((B, 1), jnp.float32),
        scratch_shapes=[
            pltpu.VMEM((B, L, DI), jnp.float32),
            pltpu.VMEM((B, L, DI), jnp.float32),
            pltpu.VMEM((B, L, 2 * DS), jnp.float32),
            pltpu.VMEM((B, L, DI), jnp.float32),
            pltpu.VMEM((B, L, DI), jnp.float32),
            pltpu.VMEM((B, L, 2 * DS), jnp.float32),
        ],
        interpret=_INTERP,
    )(emb, nt2, jnp.reshape(cls_token, (1, DM)), normw2, in_proj_W, conv_w,
      cb2, x_proj_W, dt_w, dtb2, A_log, dssm2, out_proj_W, head_w, hb2)

    return jnp.reshape(out, (B,))


# speedup vs baseline: 19.0737x; 1.1553x over previous
import jax
import jax.numpy as jnp
from jax.experimental import pallas as pl
from jax.experimental.pallas import tpu as pltpu

B, T, WMAX, DM = 4, 512, 50, 200
DI, DS, DTR, K = 400, 16, 13, 4
POS = T // 2
L = T + 1

TC_AGG = 128

_INTERP = False


def _sigmoid(x):
    return 1.0 / (1.0 + jnp.exp(-x))


def _silu(x):
    return x * _sigmoid(x)


def _softplus(x):
    return jnp.maximum(x, 0.0) + jnp.log(1.0 + jnp.exp(-jnp.abs(x)))


def _agg_body(nw_ref, ids_ref, wa_ref, ba_ref, out_ref):
    ids = ids_ref[0]
    wa = wa_ref[:, 0]
    scores = jnp.sum(ids * wa[None, None, :], axis=-1) + ba_ref[0, 0]
    nw = nw_ref[0, 0]
    wm = jax.lax.broadcasted_iota(jnp.int32, (TC_AGG, WMAX), 1) < nw[:, None]
    scores = jnp.where(wm, scores, -1e30)
    m = jnp.max(scores, axis=1, keepdims=True)
    e = jnp.exp(scores - m)
    attn = e / jnp.sum(e, axis=1, keepdims=True)
    attn = jnp.where(wm, attn, 0.0)
    out_ref[0] = jnp.sum(attn[:, :, None] * ids, axis=1)


def _mamba_body(emb_ref, nt_ref, cls_ref, normw_ref, inW_ref, convw_ref,
                convb_ref, xW_ref, dtw_ref, dtb_ref, Alog_ref, Dssm_ref,
                outW_ref, headw_ref, headb_ref, out_ref):
    convw = convw_ref[...]
    convb = convb_ref[0][None, :]
    dtb = dtb_ref[0][None, :]
    normw = normw_ref[0][None, :]
    cls = cls_ref[...]
    AT = -jnp.exp(jnp.transpose(Alog_ref[...]))

    W = POS + 1
    ri = jax.lax.broadcasted_iota(jnp.int32, (W, W), 0)
    ci = jax.lax.broadcasted_iota(jnp.int32, (W, W), 1)
    TF = jnp.where(ci > ri, 1.0, 0.0)
    TB = jnp.where(ci < ri, 1.0, 0.0)

    grows = []
    z_pos = []
    for b in range(B):
        nt_b = jnp.broadcast_to(nt_ref[0:1, b:b + 1], (T, DM))
        emb_b = emb_ref[b]
        tmask = jax.lax.broadcasted_iota(jnp.int32, (T, DM), 0) < nt_b
        x0 = jnp.where(tmask, emb_b, 0.0)
        x_b = jnp.concatenate([x0[:POS], cls, x0[POS:]], axis=0)
        h_b = x_b * jax.lax.rsqrt(
            jnp.mean(x_b * x_b, axis=-1, keepdims=True) + 1e-5) * normw
        xz_b = h_b @ inW_ref[...]
        xs0 = xz_b[:, :DI]
        z_pos.append(xz_b[POS:POS + 1, DI:])

        zpad = jnp.zeros((K - 1, DI), jnp.float32)
        xpF = jnp.concatenate([zpad, xs0], axis=0)
        xpB = jnp.concatenate([xs0, zpad], axis=0)
        accF = convb
        accB = convb
        for k in range(K):
            wk = convw[:, k][None, :]
            accF = accF + xpF[k:k + L] * wk
            accB = accB + xpB[K - 1 - k:K - 1 - k + L] * wk
        xsF_b = _silu(accF)
        xsB_b = _silu(accB)

        dssm = Dssm_ref[0][None, :]
        ysum = jnp.zeros((1, DI), jnp.float32)
        for xs_b, TRI, w0, pos_row in (
                (xsF_b, TF, 0, W - 1),
                (xsB_b, TB, POS, 0)):
            dbc_b = xs_b @ xW_ref[...]
            dt_b = _softplus(dbc_b[:, :DTR] @ dtw_ref[...] + dtb)
            dtw_w = dt_b[w0:w0 + W]
            duw = dtw_w * xs_b[w0:w0 + W]
            bmw = dbc_b[w0:w0 + W, DTR:DTR + DS]
            cs = TRI @ dtw_w
            y = jnp.zeros((1, DI), jnp.float32)
            for s in range(DS):
                cms = dbc_b[w0 + pos_row:w0 + pos_row + 1,
                            DTR + DS + s:DTR + DS + s + 1]
                wcol = bmw[:, s:s + 1] * cms
                contrib = jnp.exp(cs * AT[s:s + 1, :]) * duw * wcol
                y = y + jnp.sum(contrib, axis=0, keepdims=True)
            ysum = ysum + y + xs_b[POS:POS + 1] * dssm
        grows.append(ysum * _silu(z_pos[b]))
    G = jnp.concatenate(grows, axis=0)
    outp = G @ outW_ref[...]
    xfin = jnp.broadcast_to(cls, (B, DM)) + outp
    logits = xfin @ headw_ref[...] + jnp.broadcast_to(headb_ref[0:1, 0:1], (B, 1))
    out_ref[...] = _sigmoid(logits)


def kernel(input_ids, cls_token, W_attn, b_attn, norm_w, in_proj_W, conv_w,
           conv_b, x_proj_W, dt_w, dt_b, A_log, D_ssm, out_proj_W, head_w,
           head_b, n_tweets, n_words):
    nw3 = jnp.reshape(n_words, (B, 1, T)).astype(jnp.int32)
    ba2 = jnp.reshape(b_attn, (1, 1))
    nt2 = jnp.reshape(n_tweets, (1, B)).astype(jnp.int32)
    cb2 = jnp.reshape(conv_b, (1, DI))
    dtb2 = jnp.reshape(dt_b, (1, DI))
    dssm2 = jnp.reshape(D_ssm, (1, DI))
    normw2 = jnp.reshape(norm_w, (1, DM))
    hb2 = jnp.reshape(head_b, (1, 1))

    emb = pl.pallas_call(
        _agg_body,
        grid=(B, T // TC_AGG),
        in_specs=[
            pl.BlockSpec((1, 1, TC_AGG), lambda b, t: (b, 0, t)),
            pl.BlockSpec((1, TC_AGG, WMAX, DM), lambda b, t: (b, t, 0, 0)),
            pl.BlockSpec((DM, 1), lambda b, t: (0, 0)),
            pl.BlockSpec((1, 1), lambda b, t: (0, 0)),
        ],
        out_specs=pl.BlockSpec((1, TC_AGG, DM), lambda b, t: (b, t, 0)),
        out_shape=jax.ShapeDtypeStruct((B, T, DM), jnp.float32),
        interpret=_INTERP,
    )(nw3, input_ids, W_attn, ba2)

    out = pl.pallas_call(
        _mamba_body,
        out_shape=jax.ShapeDtypeStruct((B, 1), jnp.float32),
        interpret=_INTERP,
    )(emb, nt2, jnp.reshape(cls_token, (1, DM)), normw2, in_proj_W, conv_w,
      cb2, x_proj_W, dt_w, dtb2, A_log, dssm2, out_proj_W, head_w, hb2)

    return jnp.reshape(out, (B,))


# speedup vs baseline: 19.6072x; 1.0280x over previous
import jax
import jax.numpy as jnp
from jax.experimental import pallas as pl
from jax.experimental.pallas import tpu as pltpu

B, T, WMAX, DM = 4, 512, 50, 200
DI, DS, DTR, K = 400, 16, 13, 4
POS = T // 2
L = T + 1

TC_AGG = 256

_INTERP = False


def _sigmoid(x):
    return 1.0 / (1.0 + jnp.exp(-x))


def _silu(x):
    return x * _sigmoid(x)


def _softplus(x):
    return jnp.maximum(x, 0.0) + jnp.log(1.0 + jnp.exp(-jnp.abs(x)))


def _agg_body(nw_ref, ids_ref, w50_ref, r_ref, s_ref, ba_ref, out_ref):
    ids = ids_ref[0]
    scores = ids @ w50_ref[...] + jnp.broadcast_to(ba_ref[0:1, 0:1],
                                                   (TC_AGG, WMAX))
    nw = nw_ref[0, 0]
    wm = jax.lax.broadcasted_iota(jnp.int32, (TC_AGG, WMAX), 1) < nw[:, None]
    scores = jnp.where(wm, scores, -1e30)
    m = jnp.max(scores, axis=1, keepdims=True)
    e = jnp.exp(scores - m)
    attn = e / jnp.sum(e, axis=1, keepdims=True)
    attn = jnp.where(wm, attn, 0.0)
    attn_e = attn @ r_ref[...]
    out_ref[0] = (ids * attn_e) @ s_ref[...]


def _mamba_body(emb_ref, nt_ref, cls_ref, normw_ref, inW_ref, convw_ref,
                convb_ref, xW_ref, dtw_ref, dtb_ref, Alog_ref, Dssm_ref,
                outW_ref, headw_ref, headb_ref, out_ref):
    convw = convw_ref[...]
    convb = convb_ref[0][None, :]
    dtb = dtb_ref[0][None, :]
    normw = normw_ref[0][None, :]
    cls = cls_ref[...]
    AT = -jnp.exp(jnp.transpose(Alog_ref[...]))

    W = POS + 1
    ri = jax.lax.broadcasted_iota(jnp.int32, (W, W), 0)
    ci = jax.lax.broadcasted_iota(jnp.int32, (W, W), 1)
    TF = jnp.where(ci > ri, 1.0, 0.0)
    TB = jnp.where(ci < ri, 1.0, 0.0)

    grows = []
    z_pos = []
    for b in range(B):
        nt_b = jnp.broadcast_to(nt_ref[0:1, b:b + 1], (T, DM))
        emb_b = emb_ref[b]
        tmask = jax.lax.broadcasted_iota(jnp.int32, (T, DM), 0) < nt_b
        x0 = jnp.where(tmask, emb_b, 0.0)
        x_b = jnp.concatenate([x0[:POS], cls, x0[POS:]], axis=0)
        h_b = x_b * jax.lax.rsqrt(
            jnp.mean(x_b * x_b, axis=-1, keepdims=True) + 1e-5) * normw
        xz_b = h_b @ inW_ref[...]
        xs0 = xz_b[:, :DI]
        z_pos.append(xz_b[POS:POS + 1, DI:])

        zpad = jnp.zeros((K - 1, DI), jnp.float32)
        xpF = jnp.concatenate([zpad, xs0], axis=0)
        xpB = jnp.concatenate([xs0, zpad], axis=0)
        accF = convb
        accB = convb
        for k in range(K):
            wk = convw[:, k][None, :]
            accF = accF + xpF[k:k + L] * wk
            accB = accB + xpB[K - 1 - k:K - 1 - k + L] * wk
        xsF_b = _silu(accF)
        xsB_b = _silu(accB)

        dssm = Dssm_ref[0][None, :]
        ysum = jnp.zeros((1, DI), jnp.float32)
        for xs_b, TRI, w0, pos_row in (
                (xsF_b, TF, 0, W - 1),
                (xsB_b, TB, POS, 0)):
            dbc_b = xs_b @ xW_ref[...]
            dt_b = _softplus(dbc_b[:, :DTR] @ dtw_ref[...] + dtb)
            dtw_w = dt_b[w0:w0 + W]
            duw = dtw_w * xs_b[w0:w0 + W]
            bmw = dbc_b[w0:w0 + W, DTR:DTR + DS]
            cs = TRI @ dtw_w
            y = jnp.zeros((1, DI), jnp.float32)
            for s in range(DS):
                cms = dbc_b[w0 + pos_row:w0 + pos_row + 1,
                            DTR + DS + s:DTR + DS + s + 1]
                wcol = bmw[:, s:s + 1] * cms
                contrib = jnp.exp(cs * AT[s:s + 1, :]) * duw * wcol
                y = y + jnp.sum(contrib, axis=0, keepdims=True)
            ysum = ysum + y + xs_b[POS:POS + 1] * dssm
        grows.append(ysum * _silu(z_pos[b]))
    G = jnp.concatenate(grows, axis=0)
    outp = G @ outW_ref[...]
    xfin = jnp.broadcast_to(cls, (B, DM)) + outp
    logits = xfin @ headw_ref[...] + jnp.broadcast_to(headb_ref[0:1, 0:1], (B, 1))
    out_ref[...] = _sigmoid(logits)


def kernel(input_ids, cls_token, W_attn, b_attn, norm_w, in_proj_W, conv_w,
           conv_b, x_proj_W, dt_w, dt_b, A_log, D_ssm, out_proj_W, head_w,
           head_b, n_tweets, n_words):
    nw3 = jnp.reshape(n_words, (B, 1, T)).astype(jnp.int32)
    ba2 = jnp.reshape(b_attn, (1, 1))
    nt2 = jnp.reshape(n_tweets, (1, B)).astype(jnp.int32)
    cb2 = jnp.reshape(conv_b, (1, DI))
    dtb2 = jnp.reshape(dt_b, (1, DI))
    dssm2 = jnp.reshape(D_ssm, (1, DI))
    normw2 = jnp.reshape(norm_w, (1, DM))
    hb2 = jnp.reshape(head_b, (1, 1))

    ids2 = jnp.reshape(input_ids, (B, T, WMAX * DM))
    eyew = jnp.eye(WMAX, dtype=jnp.float32)
    w50 = jnp.reshape(eyew[:, None, :] * W_attn[:, 0][None, :, None],
                      (WMAX * DM, WMAX))
    rexp = jnp.reshape(jnp.broadcast_to(eyew[:, :, None], (WMAX, WMAX, DM)),
                       (WMAX, WMAX * DM))
    ssum = jnp.reshape(jnp.broadcast_to(jnp.eye(DM, dtype=jnp.float32)[None],
                                        (WMAX, DM, DM)), (WMAX * DM, DM))
    emb = pl.pallas_call(
        _agg_body,
        grid=(B, T // TC_AGG),
        in_specs=[
            pl.BlockSpec((1, 1, TC_AGG), lambda b, t: (b, 0, t)),
            pl.BlockSpec((1, TC_AGG, WMAX * DM), lambda b, t: (b, t, 0)),
            pl.BlockSpec((WMAX * DM, WMAX), lambda b, t: (0, 0)),
            pl.BlockSpec((WMAX, WMAX * DM), lambda b, t: (0, 0)),
            pl.BlockSpec((WMAX * DM, DM), lambda b, t: (0, 0)),
            pl.BlockSpec((1, 1), lambda b, t: (0, 0)),
        ],
        out_specs=pl.BlockSpec((1, TC_AGG, DM), lambda b, t: (b, t, 0)),
        out_shape=jax.ShapeDtypeStruct((B, T, DM), jnp.float32),
        interpret=_INTERP,
    )(nw3, ids2, w50, rexp, ssum, ba2)

    out = pl.pallas_call(
        _mamba_body,
        out_shape=jax.ShapeDtypeStruct((B, 1), jnp.float32),
        interpret=_INTERP,
    )(emb, nt2, jnp.reshape(cls_token, (1, DM)), normw2, in_proj_W, conv_w,
      cb2, x_proj_W, dt_w, dtb2, A_log, dssm2, out_proj_W, head_w, hb2)

    return jnp.reshape(out, (B,))


# speedup vs baseline: 20.0978x; 1.0250x over previous
import jax
import jax.numpy as jnp
from jax.experimental import pallas as pl
from jax.experimental.pallas import tpu as pltpu

B, T, WMAX, DM = 4, 512, 50, 200
DI, DS, DTR, K = 400, 16, 13, 4
POS = T // 2
L = T + 1

TC_AGG = 256

_INTERP = False


def _sigmoid(x):
    return 1.0 / (1.0 + jnp.exp(-x))


def _silu(x):
    return x * _sigmoid(x)


def _softplus(x):
    return jnp.maximum(x, 0.0) + jnp.log(1.0 + jnp.exp(-jnp.abs(x)))


def _agg_body(nw_ref, ids_ref, w50_ref, r_ref, s_ref, ba_ref, out_ref):
    ids = ids_ref[0].astype(jnp.bfloat16)
    scores = jax.lax.dot_general(
        ids, w50_ref[...], (((1,), (0,)), ((), ())),
        preferred_element_type=jnp.float32)
    scores = scores + jnp.broadcast_to(ba_ref[0:1, 0:1], (TC_AGG, WMAX))
    nw = nw_ref[0, 0]
    wm = jax.lax.broadcasted_iota(jnp.int32, (TC_AGG, WMAX), 1) < nw[:, None]
    scores = jnp.where(wm, scores, -1e30)
    m = jnp.max(scores, axis=1, keepdims=True)
    e = jnp.exp(scores - m)
    attn = e / jnp.sum(e, axis=1, keepdims=True)
    attn = jnp.where(wm, attn, 0.0).astype(jnp.bfloat16)
    attn_e = jax.lax.dot_general(
        attn, r_ref[...], (((1,), (0,)), ((), ())),
        preferred_element_type=jnp.float32).astype(jnp.bfloat16)
    out_ref[0] = jax.lax.dot_general(
        ids * attn_e, s_ref[...], (((1,), (0,)), ((), ())),
        preferred_element_type=jnp.float32)


def _mamba_body(emb_ref, nt_ref, cls_ref, normw_ref, inW_ref, convw_ref,
                convb_ref, xW_ref, dtw_ref, dtb_ref, Alog_ref, Dssm_ref,
                outW_ref, headw_ref, headb_ref, out_ref):
    convw = convw_ref[...]
    convb = convb_ref[0][None, :]
    dtb = dtb_ref[0][None, :]
    normw = normw_ref[0][None, :]
    cls = cls_ref[...]
    AT = -jnp.exp(jnp.transpose(Alog_ref[...]))

    W = POS + 1
    ri = jax.lax.broadcasted_iota(jnp.int32, (W, W), 0)
    ci = jax.lax.broadcasted_iota(jnp.int32, (W, W), 1)
    TF = jnp.where(ci > ri, 1.0, 0.0)
    TB = jnp.where(ci < ri, 1.0, 0.0)

    grows = []
    z_pos = []
    for b in range(B):
        nt_b = jnp.broadcast_to(nt_ref[0:1, b:b + 1], (T, DM))
        emb_b = emb_ref[b]
        tmask = jax.lax.broadcasted_iota(jnp.int32, (T, DM), 0) < nt_b
        x0 = jnp.where(tmask, emb_b, 0.0)
        x_b = jnp.concatenate([x0[:POS], cls, x0[POS:]], axis=0)
        h_b = x_b * jax.lax.rsqrt(
            jnp.mean(x_b * x_b, axis=-1, keepdims=True) + 1e-5) * normw
        xz_b = h_b @ inW_ref[...]
        xs0 = xz_b[:, :DI]
        z_pos.append(xz_b[POS:POS + 1, DI:])

        zpad = jnp.zeros((K - 1, DI), jnp.float32)
        xpF = jnp.concatenate([zpad, xs0], axis=0)
        xpB = jnp.concatenate([xs0, zpad], axis=0)
        accF = convb
        accB = convb
        for k in range(K):
            wk = convw[:, k][None, :]
            accF = accF + xpF[k:k + L] * wk
            accB = accB + xpB[K - 1 - k:K - 1 - k + L] * wk
        xsF_b = _silu(accF)
        xsB_b = _silu(accB)

        dssm = Dssm_ref[0][None, :]
        ysum = jnp.zeros((1, DI), jnp.float32)
        for xs_b, TRI, w0, pos_row in (
                (xsF_b, TF, 0, W - 1),
                (xsB_b, TB, POS, 0)):
            dbc_b = xs_b @ xW_ref[...]
            dt_b = _softplus(dbc_b[:, :DTR] @ dtw_ref[...] + dtb)
            dtw_w = dt_b[w0:w0 + W]
            duw = dtw_w * xs_b[w0:w0 + W]
            bmw = dbc_b[w0:w0 + W, DTR:DTR + DS]
            cs = TRI @ dtw_w
            y = jnp.zeros((1, DI), jnp.float32)
            for s in range(DS):
                cms = dbc_b[w0 + pos_row:w0 + pos_row + 1,
                            DTR + DS + s:DTR + DS + s + 1]
                wcol = bmw[:, s:s + 1] * cms
                contrib = jnp.exp(cs * AT[s:s + 1, :]) * duw * wcol
                y = y + jnp.sum(contrib, axis=0, keepdims=True)
            ysum = ysum + y + xs_b[POS:POS + 1] * dssm
        grows.append(ysum * _silu(z_pos[b]))
    G = jnp.concatenate(grows, axis=0)
    outp = G @ outW_ref[...]
    xfin = jnp.broadcast_to(cls, (B, DM)) + outp
    logits = xfin @ headw_ref[...] + jnp.broadcast_to(headb_ref[0:1, 0:1], (B, 1))
    out_ref[...] = _sigmoid(logits)


def kernel(input_ids, cls_token, W_attn, b_attn, norm_w, in_proj_W, conv_w,
           conv_b, x_proj_W, dt_w, dt_b, A_log, D_ssm, out_proj_W, head_w,
           head_b, n_tweets, n_words):
    nw3 = jnp.reshape(n_words, (B, 1, T)).astype(jnp.int32)
    ba2 = jnp.reshape(b_attn, (1, 1))
    nt2 = jnp.reshape(n_tweets, (1, B)).astype(jnp.int32)
    cb2 = jnp.reshape(conv_b, (1, DI))
    dtb2 = jnp.reshape(dt_b, (1, DI))
    dssm2 = jnp.reshape(D_ssm, (1, DI))
    normw2 = jnp.reshape(norm_w, (1, DM))
    hb2 = jnp.reshape(head_b, (1, 1))

    ids2 = jnp.reshape(input_ids, (B, T, WMAX * DM))
    eyew = jnp.eye(WMAX, dtype=jnp.float32)
    w50 = jnp.reshape(eyew[:, None, :] * W_attn[:, 0][None, :, None],
                      (WMAX * DM, WMAX)).astype(jnp.bfloat16)
    rexp = jnp.reshape(jnp.broadcast_to(eyew[:, :, None], (WMAX, WMAX, DM)),
                       (WMAX, WMAX * DM)).astype(jnp.bfloat16)
    ssum = jnp.reshape(jnp.broadcast_to(jnp.eye(DM, dtype=jnp.float32)[None],
                                        (WMAX, DM, DM)),
                       (WMAX * DM, DM)).astype(jnp.bfloat16)
    emb = pl.pallas_call(
        _agg_body,
        grid=(B, T // TC_AGG),
        in_specs=[
            pl.BlockSpec((1, 1, TC_AGG), lambda b, t: (b, 0, t)),
            pl.BlockSpec((1, TC_AGG, WMAX * DM), lambda b, t: (b, t, 0)),
            pl.BlockSpec((WMAX * DM, WMAX), lambda b, t: (0, 0)),
            pl.BlockSpec((WMAX, WMAX * DM), lambda b, t: (0, 0)),
            pl.BlockSpec((WMAX * DM, DM), lambda b, t: (0, 0)),
            pl.BlockSpec((1, 1), lambda b, t: (0, 0)),
        ],
        out_specs=pl.BlockSpec((1, TC_AGG, DM), lambda b, t: (b, t, 0)),
        out_shape=jax.ShapeDtypeStruct((B, T, DM), jnp.float32),
        interpret=_INTERP,
    )(nw3, ids2, w50, rexp, ssum, ba2)

    out = pl.pallas_call(
        _mamba_body,
        out_shape=jax.ShapeDtypeStruct((B, 1), jnp.float32),
        interpret=_INTERP,
    )(emb, nt2, jnp.reshape(cls_token, (1, DM)), normw2, in_proj_W, conv_w,
      cb2, x_proj_W, dt_w, dtb2, A_log, dssm2, out_proj_W, head_w, hb2)

    return jnp.reshape(out, (B,))
